# baseline (speedup 1.0000x reference)
import functools

import jax
import jax.numpy as jnp
from jax import lax
from jax.experimental import pallas as pl
from jax.experimental.pallas import tpu as pltpu
from jax.experimental.pallas import tpu_sc as plsc

BATCH = 4
TOTAL_SEQ = 2048
D_MODEL = 1024
N_POS = 68
ROWS = BATCH * N_POS
CHUNK = 16
N_CHUNKS = ROWS // CHUNK


def _vbcast(x):
  return lax.broadcast(x, (16,))


def _sc_gather(table):
  mesh = plsc.VectorSubcoreMesh(core_axis_name="c", subcore_axis_name="s")

  @functools.partial(
      pl.kernel,
      mesh=mesh,
      out_type=jax.ShapeDtypeStruct((ROWS, D_MODEL), jnp.float32),
      compiler_params=pltpu.CompilerParams(
          disable_bounds_checks=True,
          disable_semaphore_checks=True,
          skip_device_barrier=True,
      ),
      scratch_types=[
          pltpu.VMEM((CHUNK,), jnp.int32),
          pltpu.VMEM((CHUNK, D_MODEL), jnp.float32),
          pltpu.SemaphoreType.DMA,
      ],
  )
  def k(table_hbm, out_hbm, idx_v, rows_v, sem):
    wid = lax.axis_index("s") * 2 + lax.axis_index("c")

    @pl.when(wid < N_CHUNKS)
    def _():
      r = _vbcast(wid * CHUNK) + lax.iota(jnp.int32, 16)
      bj = lax.div(r, _vbcast(jnp.int32(17)))
      kk = r - bj * _vbcast(jnp.int32(17))
      b = lax.div(bj, _vbcast(jnp.int32(4)))
      j = bj - b * _vbcast(jnp.int32(4))
      idx_v[...] = (
          b * _vbcast(jnp.int32(TOTAL_SEQ))
          + j * _vbcast(jnp.int32(512))
          + kk * _vbcast(jnp.int32(30))
      )
      pltpu.async_copy(table_hbm.at[idx_v], rows_v, sem).wait()
      pltpu.sync_copy(rows_v, out_hbm.at[pl.ds(wid * CHUNK, CHUNK)])

  return k(table)


def kernel(hidden_states):
  table = hidden_states.reshape(BATCH * TOTAL_SEQ, D_MODEL)
  out = _sc_gather(table)
  return out.reshape(BATCH, N_POS, D_MODEL)
